# baseline (speedup 1.0000x reference)
import functools

import jax
import jax.numpy as jnp
from jax import lax
from jax.experimental import pallas as pl
from jax.experimental.pallas import tpu as pltpu
from jax.experimental.pallas import tpu_sc as plsc

_NW = 32
_RING = 5


def kernel(x, weight):
    b, h = x.shape
    n = b * h
    d = weight.shape[1]
    b_per_w = n // _NW
    rows_per_w = b // _NW

    mesh = plsc.VectorSubcoreMesh(core_axis_name="c", subcore_axis_name="s")

    @functools.partial(
        pl.kernel,
        out_type=jax.ShapeDtypeStruct((h, d // 8, b // 128, 8, 128), weight.dtype),
        mesh=mesh,
        compiler_params=pltpu.CompilerParams(
            use_tc_tiling_on_sc=False, needs_layout_passes=False
        ),
        scratch_types=[
            pltpu.VMEM((b_per_w,), jnp.int32),
            pltpu.VMEM((h * rows_per_w,), jnp.int32),
            pltpu.VMEM((_RING, rows_per_w, d), jnp.float32),
            pltpu.VMEM((_RING, d // 8, 8, 128), jnp.float32),
            pltpu.SemaphoreType.DMA,
        ]
        + [pltpu.SemaphoreType.DMA for _ in range(2 * _RING)],
    )
    def k(w_hbm, i_hbm, o_hbm, idx_v, idxt_v, rows_v, tiles_v, isem, *sems):
        gsem = sems[:_RING]
        osem = sems[_RING:]
        wid = lax.axis_index("s") * 2 + lax.axis_index("c")
        base = wid * b_per_w
        pltpu.sync_copy(i_hbm.at[pl.ds(base, b_per_w)], idx_v)

        lanes = jax.lax.iota(jnp.int32, 16)

        @pl.loop(0, h)
        def _(t):
            for jj in range(rows_per_w // 16):
                vals = plsc.load_gather(idx_v, [(jj * 16 + lanes) * h + t])
                idxt_v[pl.ds(pl.multiple_of(t * rows_per_w, 128) + jj * 16, 16)] = vals

        def fire_gather(t, j):
            pltpu.async_copy(
                w_hbm.at[idxt_v.at[pl.ds(pl.multiple_of(t * rows_per_w, 128), rows_per_w)]],
                rows_v.at[j],
                gsem[j],
            )

        def wait_gather(j):
            pltpu.make_async_copy(
                w_hbm.at[pl.ds(0, rows_per_w)], rows_v.at[j], gsem[j]
            ).wait()

        def fire_out(t, j):
            pltpu.async_copy(tiles_v.at[j], o_hbm.at[t].at[:, wid], osem[j])

        def wait_out(j):
            pltpu.make_async_copy(
                tiles_v.at[j], o_hbm.at[0].at[:, wid], osem[j]
            ).wait()

        def transpose_block(j):
            buf = rows_v.at[j]
            tile = tiles_v.at[j]

            @plsc.parallel_loop(0, d // 8, unroll=2)
            def _(dhi):
                for dlo in range(8):
                    dd = dhi * 8 + dlo
                    for jj in range(rows_per_w // 16):
                        vals = plsc.load_gather(buf, [jj * 16 + lanes, lanes * 0 + dd])
                        tile[dhi, dlo, pl.ds(jj * 16, 16)] = vals

        rounds = h // _RING
        for j in range(_RING):
            fire_gather(j, j)

        @pl.loop(0, rounds)
        def _(r):
            for j in range(_RING):
                t = r * _RING + j
                wait_gather(j)

                @pl.when(r > 0)
                def _():
                    wait_out(j)

                transpose_block(j)
                fire_out(t, j)

                @pl.when(r < rounds - 1)
                def _():
                    fire_gather(t + _RING, j)

        for j in range(_RING):
            wait_out(j)

    out5d = k(weight, x.reshape(n).astype(jnp.int32))
    return jnp.transpose(out5d, (2, 4, 0, 1, 3)).reshape(b, h, d)


# speedup vs baseline: 1.1184x; 1.1184x over previous
import functools

import jax
import jax.numpy as jnp
from jax import lax
from jax.experimental import pallas as pl
from jax.experimental.pallas import tpu as pltpu
from jax.experimental.pallas import tpu_sc as plsc

_NW = 32
_RING = 5


def kernel(x, weight):
    b, h = x.shape
    n = b * h
    d = weight.shape[1]
    b_per_w = n // _NW
    rows_per_w = b // _NW

    mesh = plsc.VectorSubcoreMesh(core_axis_name="c", subcore_axis_name="s")

    @functools.partial(
        pl.kernel,
        out_type=jax.ShapeDtypeStruct((h, d // 8, b // 128, 8, 128), weight.dtype),
        mesh=mesh,
        compiler_params=pltpu.CompilerParams(
            use_tc_tiling_on_sc=False, needs_layout_passes=False
        ),
        scratch_types=[
            pltpu.VMEM((b_per_w,), jnp.int32),
            pltpu.VMEM((h * rows_per_w,), jnp.int32),
            pltpu.VMEM((_RING, rows_per_w, d), jnp.float32),
            pltpu.VMEM((_RING, d // 8, 8, 128), jnp.float32),
            pltpu.SemaphoreType.DMA,
        ]
        + [pltpu.SemaphoreType.DMA for _ in range(2 * _RING)],
    )
    def k(w_hbm, i_hbm, o_hbm, idx_v, idxt_v, rows_v, tiles_v, isem, *sems):
        gsem = sems[:_RING]
        osem = sems[_RING:]
        wid = lax.axis_index("s") * 2 + lax.axis_index("c")
        base = wid * b_per_w
        pltpu.sync_copy(i_hbm.at[pl.ds(base, b_per_w)], idx_v)

        lanes = jax.lax.iota(jnp.int32, 16)

        @pl.loop(0, h)
        def _(t):
            for jj in range(rows_per_w // 16):
                vals = plsc.load_gather(idx_v, [(jj * 16 + lanes) * h + t])
                idxt_v[pl.ds(pl.multiple_of(t * rows_per_w, 128) + jj * 16, 16)] = vals

        def fire_gather(t, j):
            pltpu.async_copy(
                w_hbm.at[idxt_v.at[pl.ds(pl.multiple_of(t * rows_per_w, 128), rows_per_w)]],
                rows_v.at[j],
                gsem[j],
            )

        def wait_gather(j):
            pltpu.make_async_copy(
                w_hbm.at[pl.ds(0, rows_per_w)], rows_v.at[j], gsem[j]
            ).wait()

        def fire_out(t, j):
            pltpu.async_copy(tiles_v.at[j], o_hbm.at[t].at[:, wid], osem[j])

        def wait_out(j):
            pltpu.make_async_copy(
                tiles_v.at[j], o_hbm.at[0].at[:, wid], osem[j]
            ).wait()

        def transpose_block(j):
            buf = rows_v.at[j]
            tile = tiles_v.at[j]

            @plsc.parallel_loop(0, rows_per_w)
            def _(bb):
                bvec = lanes * 0 + bb
                for g in range(d // 16):
                    dds = g * 16 + lanes
                    vals = buf[bb, pl.ds(g * 16, 16)]
                    plsc.store_scatter(tile, [dds // 8, dds % 8, bvec], vals)

        rounds = h // _RING
        for j in range(_RING):
            fire_gather(j, j)

        @pl.loop(0, rounds)
        def _(r):
            for j in range(_RING):
                t = r * _RING + j
                wait_gather(j)

                @pl.when(r > 0)
                def _():
                    wait_out(j)

                transpose_block(j)
                fire_out(t, j)

                @pl.when(r < rounds - 1)
                def _():
                    fire_gather(t + _RING, j)

        for j in range(_RING):
            wait_out(j)

    out5d = k(weight, x.reshape(n).astype(jnp.int32))
    return jnp.transpose(out5d, (2, 4, 0, 1, 3)).reshape(b, h, d)
